# baseline (speedup 1.0000x reference)
import functools
import math

import jax
import jax.numpy as jnp
from jax import lax
from jax.experimental import pallas as pl
from jax.experimental.pallas import tpu as pltpu
from jax.experimental.pallas import tpu_sc as plsc

_SMOOTH = 0.1
_PAD = 0


def _tc_scan(x, y_col, size):
    n = x.shape[0]
    rows_per_block = 32
    nb = n // rows_per_block

    def body(y_ref, x_ref, rs_ref, gv_ref, x0_ref):
        xb = x_ref[...]
        yb = y_ref[...]
        col = lax.broadcasted_iota(jnp.int32, xb.shape, 1)
        rs_ref[...] = jnp.sum(xb, axis=1, keepdims=True)
        gv_ref[...] = jnp.sum(jnp.where(col == yb, xb, 0.0), axis=1,
                              keepdims=True)
        x0_ref[...] = xb[:, _PAD:_PAD + 1]

    shape = jax.ShapeDtypeStruct((n, 1), jnp.float32)
    return pl.pallas_call(
        body,
        grid=(nb,),
        in_specs=[
            pl.BlockSpec((rows_per_block, 1), lambda i: (i, 0)),
            pl.BlockSpec((rows_per_block, size), lambda i: (i, 0)),
        ],
        out_specs=[
            pl.BlockSpec((rows_per_block, 1), lambda i: (i, 0)),
            pl.BlockSpec((rows_per_block, 1), lambda i: (i, 0)),
            pl.BlockSpec((rows_per_block, 1), lambda i: (i, 0)),
        ],
        out_shape=[shape, shape, shape],
        compiler_params=pltpu.CompilerParams(
            dimension_semantics=("arbitrary",)),
    )(y_col, x)


def _sc_combine(y, rs, gv, x0, size):
    n = y.shape[0]
    lanes = plsc.get_sparse_core_info().num_lanes
    nch = n // lanes
    conf = 1.0 - _SMOOTH
    low = _SMOOTH / (size - 2)
    c1 = conf * math.log(conf) + (size - 2) * low * math.log(low)
    mesh = plsc.VectorSubcoreMesh(core_axis_name="c", subcore_axis_name="s")

    @functools.partial(
        pl.kernel,
        mesh=mesh,
        out_type=jax.ShapeDtypeStruct((lanes,), jnp.float32),
        scratch_types=[
            pltpu.VMEM((n,), jnp.int32),
            pltpu.VMEM((n,), jnp.float32),
            pltpu.VMEM((n,), jnp.float32),
            pltpu.VMEM((n,), jnp.float32),
            pltpu.VMEM((lanes,), jnp.float32),
        ],
    )
    def sc_kernel(y_hbm, rs_hbm, gv_hbm, x0_hbm, out_hbm,
                  y_v, rs_v, gv_v, x0_v, res_v):
        pltpu.sync_copy(y_hbm, y_v)
        pltpu.sync_copy(rs_hbm, rs_v)
        pltpu.sync_copy(gv_hbm, gv_v)
        pltpu.sync_copy(x0_hbm, x0_v)
        zero = jnp.zeros((lanes,), jnp.float32)
        cnt = zero
        g = zero
        s = zero
        for c in range(nch):
            sl = pl.ds(c * lanes, lanes)
            keep = y_v[sl] != _PAD
            cnt = cnt + jnp.where(keep, 1.0, 0.0)
            g = g + jnp.where(keep, gv_v[sl], 0.0)
            s = s + jnp.where(keep, rs_v[sl] - x0_v[sl], 0.0)
        tot = c1 * cnt - (conf - low) * g - low * s
        lane = lax.iota(jnp.int32, lanes)
        dnums = lax.GatherDimensionNumbers(
            offset_dims=(), collapsed_slice_dims=(0,), start_index_map=(0,))
        sh = 1
        while sh < lanes:
            idx = lax.bitwise_xor(lane, sh)
            tot = tot + lax.gather(
                tot, idx[:, None], dimension_numbers=dnums, slice_sizes=(1,),
                mode=lax.GatherScatterMode.PROMISE_IN_BOUNDS)
            sh *= 2
        res_v[...] = tot

        @pl.when((lax.axis_index("c") == 0) & (lax.axis_index("s") == 0))
        def _publish():
            pltpu.sync_copy(res_v, out_hbm)

    return sc_kernel(y, rs, gv, x0)


def kernel(x, y):
    n, size = x.shape
    rs, gv, x0 = _tc_scan(x, y.reshape(n, 1), size)
    out = _sc_combine(y, rs.reshape(-1), gv.reshape(-1), x0.reshape(-1), size)
    return out[0]


# speedup vs baseline: 1.1762x; 1.1762x over previous
import functools
import math

import jax
import jax.numpy as jnp
from jax import lax
from jax.experimental import pallas as pl
from jax.experimental.pallas import tpu as pltpu
from jax.experimental.pallas import tpu_sc as plsc

_SMOOTH = 0.1
_PAD = 0


def _tc_scan(x, y_col, size):
    n = x.shape[0]
    rows_per_block = 256
    cols_per_block = 16000
    nb = n // rows_per_block
    nc = size // cols_per_block

    def body(y_ref, x_ref, rs_ref, gv_ref, x0_ref):
        j = pl.program_id(1)
        xb = x_ref[...]
        yb = y_ref[...]
        col = (lax.broadcasted_iota(jnp.int32, xb.shape, 1)
               + j * cols_per_block)
        rs = jnp.sum(xb, axis=1, keepdims=True)
        gv = jnp.sum(jnp.where(col == yb, xb, 0.0), axis=1, keepdims=True)

        @pl.when(j == 0)
        def _first():
            rs_ref[...] = rs
            gv_ref[...] = gv
            x0_ref[...] = xb[:, _PAD:_PAD + 1]

        @pl.when(j > 0)
        def _rest():
            rs_ref[...] += rs
            gv_ref[...] += gv

    shape = jax.ShapeDtypeStruct((n, 1), jnp.float32)
    return pl.pallas_call(
        body,
        grid=(nb, nc),
        in_specs=[
            pl.BlockSpec((rows_per_block, 1), lambda i, j: (i, 0)),
            pl.BlockSpec((rows_per_block, cols_per_block), lambda i, j: (i, j)),
        ],
        out_specs=[
            pl.BlockSpec((rows_per_block, 1), lambda i, j: (i, 0)),
            pl.BlockSpec((rows_per_block, 1), lambda i, j: (i, 0)),
            pl.BlockSpec((rows_per_block, 1), lambda i, j: (i, 0)),
        ],
        out_shape=[shape, shape, shape],
        compiler_params=pltpu.CompilerParams(
            dimension_semantics=("arbitrary", "arbitrary")),
    )(y_col, x)


def _sc_combine(y, rs, gv, x0, size):
    n = y.shape[0]
    lanes = plsc.get_sparse_core_info().num_lanes
    nch = n // lanes
    conf = 1.0 - _SMOOTH
    low = _SMOOTH / (size - 2)
    c1 = conf * math.log(conf) + (size - 2) * low * math.log(low)
    mesh = plsc.VectorSubcoreMesh(core_axis_name="c", subcore_axis_name="s")

    @functools.partial(
        pl.kernel,
        mesh=mesh,
        out_type=jax.ShapeDtypeStruct((lanes,), jnp.float32),
        scratch_types=[
            pltpu.VMEM((n,), jnp.int32),
            pltpu.VMEM((n,), jnp.float32),
            pltpu.VMEM((n,), jnp.float32),
            pltpu.VMEM((n,), jnp.float32),
            pltpu.VMEM((lanes,), jnp.float32),
        ],
    )
    def sc_kernel(y_hbm, rs_hbm, gv_hbm, x0_hbm, out_hbm,
                  y_v, rs_v, gv_v, x0_v, res_v):
        pltpu.sync_copy(y_hbm, y_v)
        pltpu.sync_copy(rs_hbm, rs_v)
        pltpu.sync_copy(gv_hbm, gv_v)
        pltpu.sync_copy(x0_hbm, x0_v)
        zero = jnp.zeros((lanes,), jnp.float32)
        cnt = zero
        g = zero
        s = zero
        for c in range(nch):
            sl = pl.ds(c * lanes, lanes)
            keep = y_v[sl] != _PAD
            cnt = cnt + jnp.where(keep, 1.0, 0.0)
            g = g + jnp.where(keep, gv_v[sl], 0.0)
            s = s + jnp.where(keep, rs_v[sl] - x0_v[sl], 0.0)
        tot = c1 * cnt - (conf - low) * g - low * s
        lane = lax.iota(jnp.int32, lanes)
        dnums = lax.GatherDimensionNumbers(
            offset_dims=(), collapsed_slice_dims=(0,), start_index_map=(0,))
        sh = 1
        while sh < lanes:
            idx = lax.bitwise_xor(lane, sh)
            tot = tot + lax.gather(
                tot, idx[:, None], dimension_numbers=dnums, slice_sizes=(1,),
                mode=lax.GatherScatterMode.PROMISE_IN_BOUNDS)
            sh *= 2
        res_v[...] = tot

        @pl.when((lax.axis_index("c") == 0) & (lax.axis_index("s") == 0))
        def _publish():
            pltpu.sync_copy(res_v, out_hbm)

    return sc_kernel(y, rs, gv, x0)


def kernel(x, y):
    n, size = x.shape
    rs, gv, x0 = _tc_scan(x, y.reshape(n, 1), size)
    out = _sc_combine(y, rs.reshape(-1), gv.reshape(-1), x0.reshape(-1), size)
    return out[0]


# speedup vs baseline: 1.1859x; 1.0082x over previous
import functools
import math

import jax
import jax.numpy as jnp
from jax import lax
from jax.experimental import pallas as pl
from jax.experimental.pallas import tpu as pltpu
from jax.experimental.pallas import tpu_sc as plsc

_SMOOTH = 0.1
_PAD = 0


def _tc_scan(x, y_col, size):
    n = x.shape[0]
    rows_per_block = 256
    cols_per_block = 16000
    nb = n // rows_per_block
    nc = size // cols_per_block

    def body(y_ref, x_ref, rs_ref, gv_ref, x0_ref):
        i = pl.program_id(0)
        j = pl.program_id(1)
        xb = x_ref[...]
        yb = y_ref[...]
        col = (lax.broadcasted_iota(jnp.int32, xb.shape, 1)
               + j * cols_per_block)
        rs = jnp.sum(xb, axis=1, keepdims=True)
        gv = jnp.sum(jnp.where(col == yb, xb, 0.0), axis=1, keepdims=True)
        row0 = pl.ds(i * rows_per_block, rows_per_block)

        @pl.when(j == 0)
        def _first():
            rs_ref[row0, :] = rs
            gv_ref[row0, :] = gv
            x0_ref[row0, :] = xb[:, _PAD:_PAD + 1]

        @pl.when(j > 0)
        def _rest():
            rs_ref[row0, :] += rs
            gv_ref[row0, :] += gv

    shape = jax.ShapeDtypeStruct((n, 1), jnp.float32)
    out_spec = pl.BlockSpec((n, 1), lambda i, j: (0, 0))
    return pl.pallas_call(
        body,
        grid=(nb, nc),
        in_specs=[
            pl.BlockSpec((rows_per_block, 1), lambda i, j: (i, 0)),
            pl.BlockSpec((rows_per_block, cols_per_block), lambda i, j: (i, j)),
        ],
        out_specs=[out_spec, out_spec, out_spec],
        out_shape=[shape, shape, shape],
        compiler_params=pltpu.CompilerParams(
            dimension_semantics=("arbitrary", "arbitrary")),
    )(y_col, x)


def _sc_combine(y, rs, gv, x0, size):
    n = y.shape[0]
    lanes = plsc.get_sparse_core_info().num_lanes
    nch = n // lanes
    conf = 1.0 - _SMOOTH
    low = _SMOOTH / (size - 2)
    c1 = conf * math.log(conf) + (size - 2) * low * math.log(low)
    mesh = plsc.VectorSubcoreMesh(core_axis_name="c", subcore_axis_name="s")

    @functools.partial(
        pl.kernel,
        mesh=mesh,
        out_type=jax.ShapeDtypeStruct((lanes,), jnp.float32),
        scratch_types=[
            pltpu.VMEM((n,), jnp.int32),
            pltpu.VMEM((n,), jnp.float32),
            pltpu.VMEM((n,), jnp.float32),
            pltpu.VMEM((n,), jnp.float32),
            pltpu.VMEM((lanes,), jnp.float32),
            pltpu.SemaphoreType.DMA,
        ],
    )
    def sc_kernel(y_hbm, rs_hbm, gv_hbm, x0_hbm, out_hbm,
                  y_v, rs_v, gv_v, x0_v, res_v, sem):
        copies = [pltpu.async_copy(src, dst, sem)
                  for src, dst in ((y_hbm, y_v), (rs_hbm, rs_v),
                                   (gv_hbm, gv_v), (x0_hbm, x0_v))]
        for c_ in copies:
            c_.wait()
        zero = jnp.zeros((lanes,), jnp.float32)
        cnt = zero
        g = zero
        s = zero
        for c in range(nch):
            sl = pl.ds(c * lanes, lanes)
            keep = y_v[sl] != _PAD
            cnt = cnt + jnp.where(keep, 1.0, 0.0)
            g = g + jnp.where(keep, gv_v[sl], 0.0)
            s = s + jnp.where(keep, rs_v[sl] - x0_v[sl], 0.0)
        tot = c1 * cnt - (conf - low) * g - low * s
        lane = lax.iota(jnp.int32, lanes)
        dnums = lax.GatherDimensionNumbers(
            offset_dims=(), collapsed_slice_dims=(0,), start_index_map=(0,))
        sh = 1
        while sh < lanes:
            idx = lax.bitwise_xor(lane, sh)
            tot = tot + lax.gather(
                tot, idx[:, None], dimension_numbers=dnums, slice_sizes=(1,),
                mode=lax.GatherScatterMode.PROMISE_IN_BOUNDS)
            sh *= 2
        res_v[...] = tot

        @pl.when((lax.axis_index("c") == 0) & (lax.axis_index("s") == 0))
        def _publish():
            pltpu.sync_copy(res_v, out_hbm)

    return sc_kernel(y, rs, gv, x0)


def kernel(x, y):
    n, size = x.shape
    rs, gv, x0 = _tc_scan(x, y.reshape(n, 1), size)
    out = _sc_combine(y, rs.reshape(-1), gv.reshape(-1), x0.reshape(-1), size)
    return out[0]
